# Initial kernel scaffold; baseline (speedup 1.0000x reference)
import jax
import jax.numpy as jnp
from jax.experimental import pallas as pl


def kernel(token_ids, embed, proj_w, scale):
    raise NotImplementedError("write your pallas kernel here")


# speedup vs baseline: 5.0139x; 5.0139x over previous
import functools

import jax
import jax.numpy as jnp
from jax import lax
from jax.experimental import pallas as pl
from jax.experimental.pallas import tpu as pltpu
from jax.experimental.pallas import tpu_sc as plsc

TABLE_SIZE = 1000000
MOD = TABLE_SIZE - 1
TRI_DIM = 64
MODEL_DIM = 1024
B, S = 4, 8192
N = B * S

NC, NS, LANES = 2, 16, 16
NW = NC * NS
CHUNK = N // NW
GATHER = 128
NGATHER = CHUNK // GATHER
LEAD = 16


def _hash16(t0, t1, t2):
    r = (t0 * jnp.int32(36313)) ^ (t1 * jnp.int32(27191)) ^ (t2 * jnp.int32(51637))
    rf = r.astype(jnp.float32) + jnp.where(
        r < 0, jnp.float32(4294967296.0), jnp.float32(0.0))
    q = (rf * jnp.float32(1.0 / MOD)).astype(jnp.int32)
    rem = r - q * jnp.int32(MOD)
    rem = jnp.where(rem < 0, rem + jnp.int32(MOD), rem)
    rem = jnp.where(rem >= jnp.int32(MOD), rem - jnp.int32(MOD), rem)
    return rem


def _gather_body(tok_hbm, embed_hbm, rows_hbm, tok_v, idx_v, rows_v, sem):
    wid = lax.axis_index("s") * NC + lax.axis_index("c")
    base = wid * CHUNK
    s0 = base & (S - 1)
    lead = pl.multiple_of(
        base - jnp.where(s0 > 0, jnp.int32(LEAD), jnp.int32(0)), LEAD)
    base = pl.multiple_of(base, CHUNK)
    pltpu.sync_copy(tok_hbm.at[pl.ds(lead, LEAD)], tok_v.at[pl.ds(0, LEAD)])
    pltpu.sync_copy(tok_hbm.at[pl.ds(base, CHUNK)], tok_v.at[pl.ds(LEAD, CHUNK)])

    n_fill = jnp.where(s0 == 0, jnp.int32(2), jnp.int32(0))
    lane = lax.iota(jnp.int32, LANES)
    for g in range(NGATHER):
        for i in range(GATHER // LANES):
            j = g * GATHER + i * LANES
            t0 = tok_v[pl.ds(LEAD + j, LANES)]
            t1 = tok_v[pl.ds(LEAD - 1 + j, LANES)]
            t2 = tok_v[pl.ds(LEAD - 2 + j, LANES)]
            idx = _hash16(t0, t1, t2)
            if j == 0:
                idx = jnp.where(lane < n_fill, jnp.int32(MOD), idx)
            idx_v[g, pl.ds(i * LANES, LANES)] = idx
    copies = [
        pltpu.make_async_copy(
            embed_hbm.at[idx_v.at[jnp.int32(g)]],
            rows_v.at[pl.ds(g * GATHER, GATHER)], sem)
        for g in range(NGATHER)
    ]
    for c in copies:
        c.start()
    for c in copies:
        c.wait()
    pltpu.sync_copy(rows_v, rows_hbm.at[pl.ds(base, CHUNK)])


_sc_gather = functools.partial(
    pl.kernel,
    out_type=jax.ShapeDtypeStruct((N, TRI_DIM), jnp.float32),
    mesh=plsc.VectorSubcoreMesh(core_axis_name="c", subcore_axis_name="s"),
    compiler_params=pltpu.CompilerParams(use_tc_tiling_on_sc=False),
    scratch_types=[
        pltpu.VMEM((LEAD + CHUNK,), jnp.int32),
        pltpu.VMEM((NGATHER, GATHER), jnp.int32),
        pltpu.VMEM((CHUNK, TRI_DIM), jnp.float32),
        pltpu.SemaphoreType.DMA,
    ],
)(_gather_body)


def _mm_body(h_ref, w_ref, o_ref):
    o_ref[...] = lax.dot_general(
        h_ref[...], w_ref[...], (((1,), (1,)), ((), ())),
        preferred_element_type=jnp.float32)


ROWS_BLK = 2048


def _project(rows, w_scaled):
    return pl.pallas_call(
        _mm_body,
        grid=(N // ROWS_BLK,),
        in_specs=[
            pl.BlockSpec((ROWS_BLK, TRI_DIM), lambda i: (i, jnp.int32(0))),
            pl.BlockSpec((MODEL_DIM, TRI_DIM),
                         lambda i: (jnp.int32(0), jnp.int32(0))),
        ],
        out_specs=pl.BlockSpec((ROWS_BLK, MODEL_DIM),
                               lambda i: (i, jnp.int32(0))),
        out_shape=jax.ShapeDtypeStruct((N, MODEL_DIM), jnp.float32),
    )(rows, w_scaled)


def kernel(token_ids, embed, proj_w, scale):
    out_dtype = jnp.result_type(embed.dtype, proj_w.dtype, scale.dtype)
    tok = token_ids.reshape(-1).astype(jnp.int32)
    w_scaled = (proj_w * scale).astype(jnp.float32)
    rows = _sc_gather(tok, embed)
    out = _project(rows, w_scaled)
    return out.reshape(B, S, MODEL_DIM).astype(out_dtype)
